# baseline (speedup 1.0000x reference)
import functools

import jax
import jax.numpy as jnp
from jax import lax
from jax.experimental import pallas as pl
from jax.experimental.pallas import tpu as pltpu
from jax.experimental.pallas import tpu_sc as plsc

N_NODES = 10000
N_EDGES = 320000
D_EDGE = 16
HIDDEN = 10
MSG = 20
HPAD = 16

L = 16
NC = 2
NS = 16
NW = NC * NS
CH = 128
NCH = 79
PT = CH * NCH
EP = NW * PT
ACC_ROWS = 10240
BLK = 4096

def _mesh():
    return plsc.VectorSubcoreMesh(
        core_axis_name="c", subcore_axis_name="s", num_cores=NC, num_subcores=NS)


TBL = 100096


_GATHER_SPEC = dict(
    out_type=jax.ShapeDtypeStruct((EP, HPAD), jnp.float32),
    compiler_params=pltpu.CompilerParams(needs_layout_passes=False),
    scratch_types=[
        pltpu.VMEM((TBL,), jnp.float32),
        pltpu.VMEM((NCH, CH), jnp.int32),
        pltpu.VMEM((CH, HPAD), jnp.float32),
    ],
)


def _gather_body(hid_hbm, src_hbm, out_hbm, tbl_v, idx_v, buf):
    wid = lax.axis_index("s") * NC + lax.axis_index("c")
    base = wid * PT
    pltpu.sync_copy(hid_hbm, tbl_v)
    pltpu.sync_copy(src_hbm.at[wid], idx_v)
    lanes = lax.iota(jnp.int32, L)
    zrow = jnp.zeros((L,), jnp.float32)

    def zinit(r, carry):
        buf[r] = zrow
        return carry

    lax.fori_loop(0, CH, zinit, 0)

    def body(j, carry):
        def group(g, carry2):
            src10 = idx_v[j, pl.ds(g * L, L)] * HIDDEN
            rows = g * L + lanes
            for h in range(HIDDEN):
                vals = plsc.load_gather(tbl_v, [src10 + h])
                plsc.store_scatter(
                    buf, [rows, jnp.full((L,), h, jnp.int32)], vals)
            return carry2

        lax.fori_loop(0, CH // L, group, carry)
        pltpu.sync_copy(buf, out_hbm.at[pl.ds(base + j * CH, CH)])
        return carry

    lax.fori_loop(0, NCH, body, 0)


@functools.cache
def _gather_sc():
    return pl.kernel(_gather_body, mesh=_mesh(), **_GATHER_SPEC)


def _dense_body(ef_ref, nb_ref, wp_ref, bp_ref, rm_ref, s2_ref, out_ref):
    bf = jnp.bfloat16
    t2 = jnp.dot(ef_ref[...].astype(bf), wp_ref[...].astype(bf),
                 preferred_element_type=jnp.float32) + bp_ref[...]
    rep = jnp.dot(nb_ref[...].astype(bf), rm_ref[...].astype(bf),
                  preferred_element_type=jnp.float32)
    out_ref[...] = jnp.dot((t2 * rep).astype(bf), s2_ref[...].astype(bf),
                           preferred_element_type=jnp.float32)


def _dense_tc(ef, neigh, wperm, bperm, rmat, s2):
    nmh = MSG * HIDDEN
    return pl.pallas_call(
        _dense_body,
        grid=(EP // BLK,),
        in_specs=[
            pl.BlockSpec((BLK, D_EDGE), lambda i: (i, 0)),
            pl.BlockSpec((BLK, HPAD), lambda i: (i, 0)),
            pl.BlockSpec((D_EDGE, nmh), lambda i: (0, 0)),
            pl.BlockSpec((1, nmh), lambda i: (0, 0)),
            pl.BlockSpec((HPAD, nmh), lambda i: (0, 0)),
            pl.BlockSpec((nmh, MSG), lambda i: (0, 0)),
        ],
        out_specs=pl.BlockSpec((BLK, MSG), lambda i: (i, 0)),
        out_shape=jax.ShapeDtypeStruct((EP, MSG), jnp.float32),
    )(ef, neigh, wperm, bperm, rmat, s2)


_SCATTER_SPEC = dict(
    out_type=jax.ShapeDtypeStruct((NC, ACC_ROWS, MSG), jnp.float32),
    compiler_params=pltpu.CompilerParams(needs_layout_passes=False),
    scratch_types=[
        pltpu.VMEM((CH,), jnp.int32),
        pltpu.VMEM((CH, MSG), jnp.float32),
        pltpu.VMEM((CH, 128), jnp.float32),
        pltpu.VMEM_SHARED((ACC_ROWS, 128), jnp.float32),
    ],
)


def _scatter_body(msgs_hbm, dst_hbm, out_hbm, idx_c, buf20, buf128, acc):
    c = lax.axis_index("c")
    s = lax.axis_index("s")
    wid = s * NC + c
    zrows = ACC_ROWS // NS
    nz = zrows // CH
    lanes = lax.iota(jnp.int32, L)
    z16 = jnp.zeros((L,), jnp.float32)
    trows = lanes // 4
    tcols = 16 + (lanes % 4)

    def zrow(r, carry):
        for q in range(8):
            buf128[r, pl.ds(q * L, L)] = z16
        return carry

    lax.fori_loop(0, CH, zrow, 0)

    def zacc(k, carry):
        pltpu.sync_copy(buf128, acc.at[pl.ds(s * zrows + k * CH, CH)])
        return carry

    lax.fori_loop(0, nz, zacc, 0)
    plsc.subcore_barrier()

    def expand_group(g, carry):
        for rr in range(4):
            r = g * 4 + rr
            rfull = jnp.full((L,), r, jnp.int32)
            v = plsc.load_gather(buf20, [rfull, lanes])
            plsc.store_scatter(buf128, [rfull, lanes], v)
        vt = plsc.load_gather(buf20, [g * 4 + trows, tcols])
        plsc.store_scatter(buf128, [g * 4 + trows, tcols], vt)
        return carry

    def body(j, carry):
        pltpu.sync_copy(dst_hbm.at[wid, j], idx_c)
        pltpu.sync_copy(msgs_hbm.at[pl.ds(wid * PT + j * CH, CH)], buf20)
        lax.fori_loop(0, CH // 4, expand_group, 0)
        pltpu.sync_copy(buf128, acc.at[idx_c], add=True)
        return carry

    lax.fori_loop(0, NCH, body, 0)
    plsc.subcore_barrier()

    def extract_group(g, carry):
        for rr in range(4):
            r = g * 4 + rr
            rfull = jnp.full((L,), r, jnp.int32)
            v = plsc.load_gather(buf128, [rfull, lanes])
            plsc.store_scatter(buf20, [rfull, lanes], v)
        vt = plsc.load_gather(buf128, [g * 4 + trows, tcols])
        plsc.store_scatter(buf20, [g * 4 + trows, tcols], vt)
        return carry

    def obody(k, carry):
        pltpu.sync_copy(acc.at[pl.ds(s * zrows + k * CH, CH)], buf128)
        lax.fori_loop(0, CH // 4, extract_group, 0)
        pltpu.sync_copy(buf20, out_hbm.at[c, pl.ds(s * zrows + k * CH, CH)])
        return carry

    lax.fori_loop(0, nz, obody, 0)


@functools.cache
def _scatter_sc():
    return pl.kernel(_scatter_body, mesh=_mesh(), **_SCATTER_SPEC)


def _combine_body(p_ref, o_ref):
    o_ref[...] = p_ref[0] + p_ref[1]


def _combine_tc(partials):
    rb = N_NODES // 5
    return pl.pallas_call(
        _combine_body,
        grid=(5,),
        in_specs=[pl.BlockSpec((NC, rb, MSG), lambda i: (0, i, 0))],
        out_specs=pl.BlockSpec((rb, MSG), lambda i: (i, 0)),
        out_shape=jax.ShapeDtypeStruct((N_NODES, MSG), jnp.float32),
    )(partials)


def kernel(node_features, edge_features, edge_index, hidden, initial, W, b):
    nmh = MSG * HIDDEN
    src = edge_index[0].astype(jnp.int32)
    dst = edge_index[1].astype(jnp.int32)
    npad = EP - N_EDGES

    hid_flat = jnp.pad(hidden.reshape(-1), (0, TBL - N_NODES * HIDDEN))
    pad_src = (jnp.arange(npad, dtype=jnp.int32) * 61) % N_NODES
    src3d = jnp.concatenate([src, pad_src]).reshape(NW, NCH, CH)
    pad_dst = N_NODES + (jnp.arange(npad, dtype=jnp.int32) % (ACC_ROWS - N_NODES))
    dst3d = jnp.concatenate([dst, pad_dst]).reshape(NW, NCH, CH)

    cols = jnp.arange(nmh, dtype=jnp.int32)
    perm = (cols % MSG) * HIDDEN + cols // MSG
    wperm = W[:, perm]
    bperm = b[perm][None, :]
    rmat = jnp.zeros((HPAD, nmh), jnp.float32).at[cols // MSG, cols].set(1.0)
    s2 = jnp.zeros((nmh, MSG), jnp.float32).at[cols, cols % MSG].set(1.0)

    neigh = _gather_sc()(hid_flat, src3d)
    msgs = _dense_tc(edge_features, neigh, wperm, bperm, rmat, s2)
    partials = _scatter_sc()(msgs, dst3d)
    return _combine_tc(partials)


# speedup vs baseline: 1.2559x; 1.2559x over previous
import functools

import jax
import jax.numpy as jnp
from jax import lax
from jax.experimental import pallas as pl
from jax.experimental.pallas import tpu as pltpu
from jax.experimental.pallas import tpu_sc as plsc

N_NODES = 10000
N_EDGES = 320000
D_EDGE = 16
HIDDEN = 10
MSG = 20
HPAD = 16

L = 16
NC = 2
NS = 16
NW = NC * NS
CH = 128
NCH = 79
PT = CH * NCH
EP = NW * PT
ACC_ROWS = 10112
BLK = 4096

def _mesh():
    return plsc.VectorSubcoreMesh(
        core_axis_name="c", subcore_axis_name="s", num_cores=NC, num_subcores=NS)


TBL = 100096


_GATHER_SPEC = dict(
    out_type=jax.ShapeDtypeStruct((EP, HPAD), jnp.float32),
    compiler_params=pltpu.CompilerParams(needs_layout_passes=False),
    scratch_types=[
        pltpu.VMEM((TBL,), jnp.float32),
        pltpu.VMEM((NCH, CH), jnp.int32),
        pltpu.VMEM((CH, HPAD), jnp.float32),
    ],
)


def _gather_body(hid_hbm, src_hbm, out_hbm, tbl_v, idx_v, buf):
    wid = lax.axis_index("s") * NC + lax.axis_index("c")
    base = wid * PT
    pltpu.sync_copy(hid_hbm, tbl_v)
    pltpu.sync_copy(src_hbm.at[wid], idx_v)
    lanes = lax.iota(jnp.int32, L)
    zrow = jnp.zeros((L,), jnp.float32)

    def zinit(r, carry):
        buf[r] = zrow
        return carry

    lax.fori_loop(0, CH, zinit, 0)

    def body(j, carry):
        def group(g, carry2):
            src10 = idx_v[j, pl.ds(g * L, L)] * HIDDEN
            rows = g * L + lanes
            for h in range(HIDDEN):
                vals = plsc.load_gather(tbl_v, [src10 + h])
                plsc.store_scatter(
                    buf, [rows, jnp.full((L,), h, jnp.int32)], vals)
            return carry2

        lax.fori_loop(0, CH // L, group, carry)
        pltpu.sync_copy(buf, out_hbm.at[pl.ds(base + j * CH, CH)])
        return carry

    lax.fori_loop(0, NCH, body, 0)


@functools.cache
def _gather_sc():
    return pl.kernel(_gather_body, mesh=_mesh(), **_GATHER_SPEC)


def _dense_body(ef_ref, nb_ref, wp_ref, bp_ref, rm_ref, s2_ref, out_ref):
    bf = jnp.bfloat16
    t2 = jnp.dot(ef_ref[...].astype(bf), wp_ref[...].astype(bf),
                 preferred_element_type=jnp.float32) + bp_ref[...]
    rep = jnp.dot(nb_ref[...].astype(bf), rm_ref[...].astype(bf),
                  preferred_element_type=jnp.float32)
    out_ref[...] = jnp.dot((t2 * rep).astype(bf), s2_ref[...].astype(bf),
                           preferred_element_type=jnp.float32)


def _dense_tc(ef, neigh, wperm, bperm, rmat, s2):
    nmh = MSG * HIDDEN
    return pl.pallas_call(
        _dense_body,
        grid=(EP // BLK,),
        in_specs=[
            pl.BlockSpec((BLK, D_EDGE), lambda i: (i, 0)),
            pl.BlockSpec((BLK, HPAD), lambda i: (i, 0)),
            pl.BlockSpec((D_EDGE, nmh), lambda i: (0, 0)),
            pl.BlockSpec((1, nmh), lambda i: (0, 0)),
            pl.BlockSpec((HPAD, nmh), lambda i: (0, 0)),
            pl.BlockSpec((nmh, MSG), lambda i: (0, 0)),
        ],
        out_specs=pl.BlockSpec((BLK, MSG), lambda i: (i, 0)),
        out_shape=jax.ShapeDtypeStruct((EP, MSG), jnp.float32),
    )(ef, neigh, wperm, bperm, rmat, s2)


_SCATTER_SPEC = dict(
    out_type=jax.ShapeDtypeStruct((NC, ACC_ROWS, MSG), jnp.float32),
    compiler_params=pltpu.CompilerParams(needs_layout_passes=False),
    scratch_types=[
        pltpu.VMEM((CH,), jnp.int32),
        pltpu.VMEM((CH,), jnp.int32),
        pltpu.VMEM((CH, MSG), jnp.float32),
        pltpu.VMEM((CH, MSG), jnp.float32),
        pltpu.VMEM((CH, 128), jnp.float32),
        pltpu.VMEM_SHARED((ACC_ROWS, 128), jnp.float32),
        pltpu.SemaphoreType.DMA,
        pltpu.SemaphoreType.DMA,
    ],
)


def _scatter_body(msgs_hbm, dst_hbm, out_hbm, idx_in0, idx_in1,
                  buf20a, buf20b, b128, acc, insem0, insem1):
    idx_in = (idx_in0, idx_in1)
    buf20 = (buf20a, buf20b)
    insem = (insem0, insem1)
    c = lax.axis_index("c")
    s = lax.axis_index("s")
    wid = s * NC + c
    zrows = ACC_ROWS // NS
    nz = zrows // CH
    ztail = zrows - nz * CH
    lanes = lax.iota(jnp.int32, L)
    z16 = jnp.zeros((L,), jnp.float32)
    trows = lanes // 4
    tcols = 16 + (lanes % 4)

    def zrow(r, carry):
        for q in range(8):
            b128[r, pl.ds(q * L, L)] = z16
        return carry

    lax.fori_loop(0, CH, zrow, 0)

    def zacc(k, carry):
        pltpu.sync_copy(b128, acc.at[pl.ds(s * zrows + k * CH, CH)])
        return carry

    lax.fori_loop(0, nz, zacc, 0)
    pltpu.sync_copy(b128.at[pl.ds(0, ztail)],
                    acc.at[pl.ds(s * zrows + nz * CH, ztail)])
    plsc.subcore_barrier()

    def start_in(j, p):
        pltpu.make_async_copy(dst_hbm.at[wid, j], idx_in[p], insem[p]).start()
        pltpu.make_async_copy(
            msgs_hbm.at[pl.ds(wid * PT + j * CH, CH)], buf20[p],
            insem[p]).start()

    def wait_in(j, p):
        pltpu.make_async_copy(dst_hbm.at[wid, j], idx_in[p], insem[p]).wait()
        pltpu.make_async_copy(
            msgs_hbm.at[pl.ds(wid * PT + j * CH, CH)], buf20[p],
            insem[p]).wait()

    def phase_block(j, p):
        wait_in(j, p)

        def expand_group(g, carry):
            for rr in range(4):
                r = g * 4 + rr
                rfull = jnp.full((L,), r, jnp.int32)
                v = plsc.load_gather(buf20[p], [rfull, lanes])
                plsc.store_scatter(b128, [rfull, lanes], v)
            vt = plsc.load_gather(buf20[p], [g * 4 + trows, tcols])
            plsc.store_scatter(b128, [g * 4 + trows, tcols], vt)
            return carry

        lax.fori_loop(0, CH // 4, expand_group, 0)
        pltpu.sync_copy(b128, acc.at[idx_in[p]], add=True)

        @pl.when(j + 2 < NCH)
        def _():
            start_in(j + 2, p)

    start_in(0, 0)
    start_in(1, 1)

    def body(j, carry):
        @pl.when(j % 2 == 0)
        def _():
            phase_block(j, 0)

        @pl.when(j % 2 == 1)
        def _():
            phase_block(j, 1)

        return carry

    lax.fori_loop(0, NCH, body, 0)
    plsc.subcore_barrier()

    def extract_group(g, carry):
        for rr in range(4):
            r = g * 4 + rr
            rfull = jnp.full((L,), r, jnp.int32)
            v = plsc.load_gather(b128, [rfull, lanes])
            plsc.store_scatter(buf20[0], [rfull, lanes], v)
        vt = plsc.load_gather(b128, [g * 4 + trows, tcols])
        plsc.store_scatter(buf20[0], [g * 4 + trows, tcols], vt)
        return carry

    def obody(k, carry):
        pltpu.sync_copy(acc.at[pl.ds(s * zrows + k * CH, CH)], b128)
        lax.fori_loop(0, CH // 4, extract_group, 0)
        pltpu.sync_copy(buf20[0], out_hbm.at[c, pl.ds(s * zrows + k * CH, CH)])
        return carry

    lax.fori_loop(0, nz, obody, 0)
    pltpu.sync_copy(acc.at[pl.ds(s * zrows + nz * CH, ztail)],
                    b128.at[pl.ds(0, ztail)])
    lax.fori_loop(0, ztail // 4, extract_group, 0)
    pltpu.sync_copy(buf20[0].at[pl.ds(0, ztail)],
                    out_hbm.at[c, pl.ds(s * zrows + nz * CH, ztail)])


@functools.cache
def _scatter_sc():
    return pl.kernel(_scatter_body, mesh=_mesh(), **_SCATTER_SPEC)


def _combine_body(p_ref, o_ref):
    o_ref[...] = p_ref[0] + p_ref[1]


def _combine_tc(partials):
    rb = N_NODES // 5
    return pl.pallas_call(
        _combine_body,
        grid=(5,),
        in_specs=[pl.BlockSpec((NC, rb, MSG), lambda i: (0, i, 0))],
        out_specs=pl.BlockSpec((rb, MSG), lambda i: (i, 0)),
        out_shape=jax.ShapeDtypeStruct((N_NODES, MSG), jnp.float32),
    )(partials)


def kernel(node_features, edge_features, edge_index, hidden, initial, W, b):
    nmh = MSG * HIDDEN
    src = edge_index[0].astype(jnp.int32)
    dst = edge_index[1].astype(jnp.int32)
    npad = EP - N_EDGES

    hid_flat = jnp.pad(hidden.reshape(-1), (0, TBL - N_NODES * HIDDEN))
    pad_src = (jnp.arange(npad, dtype=jnp.int32) * 61) % N_NODES
    src3d = jnp.concatenate([src, pad_src]).reshape(NW, NCH, CH)
    pad_dst = N_NODES + (jnp.arange(npad, dtype=jnp.int32) % (ACC_ROWS - N_NODES))
    dst3d = jnp.concatenate([dst, pad_dst]).reshape(NW, NCH, CH)

    cols = jnp.arange(nmh, dtype=jnp.int32)
    perm = (cols % MSG) * HIDDEN + cols // MSG
    wperm = W[:, perm]
    bperm = b[perm][None, :]
    rmat = jnp.zeros((HPAD, nmh), jnp.float32).at[cols // MSG, cols].set(1.0)
    s2 = jnp.zeros((nmh, MSG), jnp.float32).at[cols, cols % MSG].set(1.0)

    neigh = _gather_sc()(hid_flat, src3d)
    msgs = _dense_tc(edge_features, neigh, wperm, bperm, rmat, s2)
    partials = _scatter_sc()(msgs, dst3d)
    return _combine_tc(partials)
